# Initial kernel scaffold; baseline (speedup 1.0000x reference)
import jax
import jax.numpy as jnp
from jax.experimental import pallas as pl


def kernel(V, E, X, W1, b1, W2, b2):
    raise NotImplementedError("write your pallas kernel here")


# speedup vs baseline: 8.6844x; 8.6844x over previous
import functools

import jax
import jax.numpy as jnp
from jax import lax
from jax.experimental import pallas as pl
from jax.experimental.pallas import tpu as pltpu
from jax.experimental.pallas import tpu_sc as plsc

N_NODES = 10000
DIM = 128
N_PAD = 10240
NW = 32
CH = 128
ROWS_PER_TILE = N_PAD // 16

_mesh = plsc.VectorSubcoreMesh(core_axis_name="c", subcore_axis_name="s",
                               num_cores=2, num_subcores=16)


def _deg_body(chunks, dst_hbm, ones_hbm, z1_hbm, deg_out, idx_v, ones_v,
              tmp_v, deg_sh):
    c = lax.axis_index("c")
    s = lax.axis_index("s")
    wid = s * 2 + c
    pltpu.sync_copy(z1_hbm, tmp_v)
    pltpu.sync_copy(tmp_v, deg_sh.at[pl.ds(s * ROWS_PER_TILE, ROWS_PER_TILE)])
    pltpu.sync_copy(ones_hbm, ones_v)
    plsc.subcore_barrier()
    base = wid * (chunks * CH)

    def chunk(i, carry):
        pltpu.sync_copy(dst_hbm.at[pl.ds(base + i * CH, CH)], idx_v)
        pltpu.sync_copy(ones_v, deg_sh.at[idx_v], add=True)
        return carry

    lax.fori_loop(0, chunks, chunk, 0)
    plsc.subcore_barrier()
    sl = pl.ds(s * ROWS_PER_TILE, ROWS_PER_TILE)
    pltpu.sync_copy(deg_sh.at[sl], tmp_v)
    pltpu.sync_copy(tmp_v, deg_out.at[c, sl])


def _make_deg(chunks):
    return pl.kernel(
        functools.partial(_deg_body, chunks),
        out_type=jax.ShapeDtypeStruct((2, N_PAD), jnp.float32),
        mesh=_mesh,
        scratch_types=[
            pltpu.VMEM((CH,), jnp.int32),
            pltpu.VMEM((CH,), jnp.float32),
            pltpu.VMEM((ROWS_PER_TILE,), jnp.float32),
            pltpu.VMEM_SHARED((N_PAD,), jnp.float32),
        ],
    )


def _spmm_body(chunks, y_hbm, src_hbm, dst_hbm, z2_hbm, out0, out1,
               sidx_v, didx_v, rows_v, agg_sh, sem):
    c = lax.axis_index("c")
    s = lax.axis_index("s")
    wid = s * 2 + c
    sl = pl.ds(s * ROWS_PER_TILE, ROWS_PER_TILE)
    pltpu.sync_copy(z2_hbm, agg_sh.at[sl])
    plsc.subcore_barrier()
    base = wid * (chunks * CH)

    def chunk(i, carry):
        off = base + i * CH
        pltpu.sync_copy(src_hbm.at[pl.ds(off, CH)], sidx_v)
        pltpu.sync_copy(dst_hbm.at[pl.ds(off, CH)], didx_v)
        pltpu.async_copy(y_hbm.at[sidx_v], rows_v, sem).wait()
        pltpu.sync_copy(rows_v, agg_sh.at[didx_v], add=True)
        return carry

    lax.fori_loop(0, chunks, chunk, 0)
    plsc.subcore_barrier()

    @pl.when(c == 0)
    def _():
        pltpu.sync_copy(agg_sh.at[sl], out0.at[sl])

    @pl.when(c == 1)
    def _():
        pltpu.sync_copy(agg_sh.at[sl], out1.at[sl])


def _make_spmm(chunks):
    return pl.kernel(
        functools.partial(_spmm_body, chunks),
        out_type=(jax.ShapeDtypeStruct((N_PAD, DIM), jnp.float32),
                  jax.ShapeDtypeStruct((N_PAD, DIM), jnp.float32)),
        mesh=_mesh,
        scratch_types=[
            pltpu.VMEM((CH,), jnp.int32),
            pltpu.VMEM((CH,), jnp.int32),
            pltpu.VMEM((CH, DIM), jnp.float32),
            pltpu.VMEM_SHARED((N_PAD, DIM), jnp.float32),
            pltpu.SemaphoreType.DMA,
        ],
    )


BN = 512
GRID = N_PAD // BN


def _norm_block(deg_ref):
    d = deg_ref[0, :] + deg_ref[1, :]
    return lax.rsqrt(jnp.maximum(d, 1.0))[:, None]


def _mm1_body(deg_ref, x_ref, w_ref, y_ref):
    norm = _norm_block(deg_ref)
    y_ref[...] = jnp.dot(x_ref[...], w_ref[...],
                         preferred_element_type=jnp.float32) * norm


def _mid_body(deg_ref, a0_ref, a1_ref, b_ref, w_ref, y_ref):
    norm = _norm_block(deg_ref)
    h = jnp.maximum((a0_ref[...] + a1_ref[...]) * norm + b_ref[...], 0.0)
    y_ref[...] = jnp.dot(h, w_ref[...],
                         preferred_element_type=jnp.float32) * norm


def _fin_body(deg_ref, a0_ref, a1_ref, b_ref, o_ref):
    norm = _norm_block(deg_ref)
    o_ref[...] = (a0_ref[...] + a1_ref[...]) * norm + b_ref[...]


_deg_spec = pl.BlockSpec((2, BN), lambda i: (0, i))
_row_spec = pl.BlockSpec((BN, DIM), lambda i: (i, 0))
_full_spec = pl.BlockSpec((DIM, DIM), lambda i: (0, 0))
_bias_spec = pl.BlockSpec((1, DIM), lambda i: (0, 0))
_out_struct = jax.ShapeDtypeStruct((N_PAD, DIM), jnp.float32)

_mm1 = pl.pallas_call(
    _mm1_body, grid=(GRID,),
    in_specs=[_deg_spec, _row_spec, _full_spec],
    out_specs=_row_spec, out_shape=_out_struct)

_mid = pl.pallas_call(
    _mid_body, grid=(GRID,),
    in_specs=[_deg_spec, _row_spec, _row_spec, _bias_spec, _full_spec],
    out_specs=_row_spec, out_shape=_out_struct)

_fin = pl.pallas_call(
    _fin_body, grid=(GRID,),
    in_specs=[_deg_spec, _row_spec, _row_spec, _bias_spec],
    out_specs=_row_spec, out_shape=_out_struct)


def kernel(V, E, X, W1, b1, W2, b2):
    ne = E.shape[1]
    per_tile = -(-ne // NW)
    per_tile = -(-per_tile // CH) * CH
    ne_pad = per_tile * NW
    chunks = per_tile // CH

    pad_idx = jnp.full((ne_pad - ne,), N_PAD - 1, dtype=jnp.int32)
    src = jnp.concatenate([E[0], pad_idx])
    dst = jnp.concatenate([E[1], pad_idx])
    x_pad = jnp.zeros((N_PAD, DIM), X.dtype).at[:N_NODES].set(X)

    ones_ch = jnp.ones((CH,), jnp.float32)
    z1 = jnp.zeros((ROWS_PER_TILE,), jnp.float32)
    z2 = jnp.zeros((ROWS_PER_TILE, DIM), jnp.float32)
    b1r = b1.reshape(1, DIM)
    b2r = b2.reshape(1, DIM)

    deg = _make_deg(chunks)(dst, ones_ch, z1)
    y1 = _mm1(deg, x_pad, W1)
    spmm = _make_spmm(chunks)
    a10, a11 = spmm(y1, src, dst, z2)
    y2 = _mid(deg, a10, a11, b1r, W2)
    a20, a21 = spmm(y2, src, dst, z2)
    out = _fin(deg, a20, a21, b2r)
    return out[:N_NODES]
